# baseline (speedup 1.0000x reference)
import functools

import jax
import jax.numpy as jnp
from jax import lax
from jax.experimental import pallas as pl
from jax.experimental.pallas import tpu as pltpu
from jax.experimental.pallas import tpu_sc as plsc

N = 1000
E = 500
D = 64
L = 16
EP = 512
NB = 4
EB = 128
NCH = D // L
DEGP = 1008


def _rsqrt16(d):
    i = plsc.bitcast(d, jnp.int32)
    i = jnp.int32(0x5F3759DF) - lax.shift_right_logical(i, 1)
    y = plsc.bitcast(i, jnp.float32)
    for _ in range(3):
        y = y * (1.5 - 0.5 * d * y * y)
    return y


def _build(interpret=False):
    mesh = plsc.VectorSubcoreMesh(
        core_axis_name="c", subcore_axis_name="s", num_cores=2, num_subcores=16
    )

    @functools.partial(
        pl.kernel,
        out_type=jax.ShapeDtypeStruct((EP,), jnp.float32),
        mesh=mesh,
        scratch_types=[
            pltpu.VMEM((N, L), jnp.float32),
            pltpu.VMEM((N, L), jnp.float32),
            pltpu.VMEM((N, L), jnp.float32),
            pltpu.VMEM((N, L), jnp.float32),
            pltpu.VMEM((N * L,), jnp.float32),
            pltpu.VMEM((EP, L), jnp.float32),
            pltpu.VMEM((EP, L), jnp.float32),
            pltpu.VMEM((DEGP,), jnp.float32),
            pltpu.VMEM((EP,), jnp.float32),
            pltpu.VMEM((EP,), jnp.int32),
            pltpu.VMEM((EP,), jnp.int32),
            pltpu.VMEM((4 * L,), jnp.float32),
            pltpu.VMEM((N, L), jnp.float32),
            pltpu.VMEM((EP,), jnp.float32),
            pltpu.VMEM((EP,), jnp.float32),
            pltpu.SemaphoreType.DMA,
            pltpu.SemaphoreType.DMA,
            pltpu.VMEM_SHARED((NCH, 2, N, L), jnp.float32),
            pltpu.VMEM_SHARED((NCH, EP), jnp.float32),
        ],
        compiler_params=pltpu.CompilerParams(needs_layout_passes=False, use_tc_tiling_on_sc=False),
        interpret=interpret,
    )
    def gcn_kernel(ei_hbm, wc_hbm, alpha_hbm, out_hbm,
                   x0v, s1v, s2v, s3v, out_f, rows, ewb,
                   deg, ew, rowi, coli, alv, zv, part, tmp, semS, semZ,
                   xsp, shared):
        cid = lax.axis_index("c")
        sid = lax.axis_index("s")
        active = jnp.logical_and(cid == 0, sid < NCH)

        @pl.when(active)
        def _work():
            chunk = sid
            pltpu.sync_copy(ei_hbm.at[0], rowi)
            pltpu.sync_copy(ei_hbm.at[1], coli)
            pltpu.sync_copy(wc_hbm.at[chunk], x0v)
            pltpu.sync_copy(wc_hbm.at[chunk], xsp.at[chunk, 0])
            pltpu.sync_copy(alpha_hbm, alv)

            zero16 = jnp.zeros((L,), jnp.float32)
            iota = lax.iota(jnp.int32, L)

            @plsc.parallel_loop(0, N, unroll=8)
            def zero_zv(i):
                zv[i, :] = zero16

            pltpu.sync_copy(zv, xsp.at[chunk, 1])

            @plsc.parallel_loop(0, DEGP // L, unroll=4)
            def zero_deg(i):
                deg[pl.ds(i * L, L)] = zero16

            @plsc.parallel_loop(0, EP // L, unroll=2)
            def deg_scatter(g):
                cv = coli[pl.ds(g * L, L)]
                valid = jnp.where(g * L + iota < E, 1.0, 0.0)
                plsc.addupdate_scatter(deg, [cv], valid)

            @plsc.parallel_loop(0, DEGP // L, unroll=2)
            def inv_sqrt(i):
                d = deg[pl.ds(i * L, L)]
                y = _rsqrt16(d)
                deg[pl.ds(i * L, L)] = jnp.where(d > 0.0, y, 0.0)

            @plsc.parallel_loop(0, EP // L, unroll=2)
            def edge_w(g):
                rv = rowi[pl.ds(g * L, L)]
                cv = coli[pl.ds(g * L, L)]
                a = plsc.load_gather(deg, [rv])
                b = plsc.load_gather(deg, [cv])
                valid = jnp.where(g * L + iota < E, 1.0, 0.0)
                ew[pl.ds(g * L, L)] = a * b * valid

            @plsc.parallel_loop(0, EP // L)
            def bcast(g):
                ewg = ew[pl.ds(g * L, L)]
                for t in range(L):
                    ewb[g * L + t, :] = jnp.broadcast_to(ewg[t], (L,))

            for k, skv, zsv in ((1, s1v, s2v), (2, s2v, s3v), (3, s3v, None)):
                srcr = (k - 1) % 2
                dstr = k % 2
                pltpu.sync_copy(xsp.at[chunk, srcr].at[rowi], rows)

                @plsc.parallel_loop(0, EP, unroll=4)
                def scale(e):
                    rows[e, :] = rows[e, :] * ewb[e, :]

                pltpu.sync_copy(rows, xsp.at[chunk, dstr].at[coli],
                                add=True)

                pltpu.sync_copy(xsp.at[chunk, dstr], skv)
                if zsv is not None:
                    pltpu.sync_copy(zsv, xsp.at[chunk, srcr])

            a0 = alv[pl.ds(0, L)]
            a1 = alv[pl.ds(L, L)]
            a2 = alv[pl.ds(2 * L, L)]
            a3 = alv[pl.ds(3 * L, L)]

            @plsc.parallel_loop(0, N, unroll=2)
            def combine(i):
                out_f[pl.ds(i * L, L)] = (a0 * x0v[i, :] + a1 * s1v[i, :]
                                          + a2 * s2v[i, :] + a3 * s3v[i, :])

            @plsc.parallel_loop(0, EP // L)
            def dot(g):
                rv16 = rowi[pl.ds(g * L, L)] * L
                cv16 = coli[pl.ds(g * L, L)] * L
                acc = zero16
                for d in range(L):
                    acc = acc + (plsc.load_gather(out_f, [rv16 + d])
                                 * plsc.load_gather(out_f, [cv16 + d]))
                part[pl.ds(g * L, L)] = acc

            pltpu.sync_copy(part, shared.at[chunk])

        plsc.subcore_barrier()

        @pl.when(jnp.logical_and(cid == 0, sid == 0))
        def _reduce():
            for t in range(1, NCH):
                pltpu.sync_copy(shared.at[t], tmp)

                @plsc.parallel_loop(0, EP // L, unroll=2)
                def accum(g, t=t):
                    part[pl.ds(g * L, L)] = (part[pl.ds(g * L, L)]
                                             + tmp[pl.ds(g * L, L)])
            pltpu.sync_copy(part, out_hbm)

    return gcn_kernel


_gcn_cache = []


def _gcn(*args):
    if not _gcn_cache:
        _gcn_cache.append(_build())
    return _gcn_cache[0](*args)


def kernel(L_edge_index_, L_self_modules_embedding_parameters_weight_,
           L_self_buffers_alpha_):
    ei = L_edge_index_
    w = L_self_modules_embedding_parameters_weight_
    alpha = L_self_buffers_alpha_
    ei_p = jnp.pad(ei.astype(jnp.int32), ((0, 0), (0, EP - E)))
    wc = w.reshape(N, NCH, L).transpose(1, 0, 2)
    alpha_p = jnp.tile(alpha.astype(jnp.float32)[:, None], (1, L)).reshape(
        4 * L)
    res = _gcn(ei_p, wc, alpha_p)
    return (res[:E],)
